# baseline (speedup 1.0000x reference)
import functools

import jax
import jax.numpy as jnp
from jax import lax
from jax.experimental import pallas as pl
from jax.experimental.pallas import tpu as pltpu
from jax.experimental.pallas import tpu_sc as plsc

N = 2_000_000
G = 128
GG = G * G
NC, NS = 2, 16
NW = NC * NS
L = 16
CH = 1024
IDXB = 128
NB = CH // IDXB
GROUPS = CH // L
NCHUNKS = (N + CH - 1) // CH
TPW = (NCHUNKS + NW - 1) // NW

_OFFS = (0, 1, G, G + 1, GG, GG + 1, GG + G, GG + G + 1)
_DIJK = ((0, 0, 0), (0, 0, 1), (0, 1, 0), (0, 1, 1),
         (1, 0, 0), (1, 0, 1), (1, 1, 0), (1, 1, 1))


def _body(theta4, coords, out, coords_v, idx_v, rows_v, w_v, out_v, sem):
    wid = lax.axis_index("s") * NC + lax.axis_index("c")
    lanes = lax.iota(jnp.int32, L)

    def chunk_body(t, carry):
        cid = wid + t * NW

        @pl.when(cid < NCHUNKS)
        def _():
            start = jnp.minimum(cid * CH, N - CH)
            pltpu.sync_copy(coords.at[pl.ds(start * 3, CH * 3)], coords_v)

            def pass_a(g, c):
                p = g * L
                pidx = p + lanes
                cbase = 3 * pidx
                x = plsc.load_gather(coords_v, [cbase]) * float(G - 1)
                y = plsc.load_gather(coords_v, [cbase + 1]) * float(G - 1)
                z = plsc.load_gather(coords_v, [cbase + 2]) * float(G - 1)
                i0 = x.astype(jnp.int32)
                j0 = y.astype(jnp.int32)
                k0 = z.astype(jnp.int32)
                w_v[0, pl.ds(p, L)] = x - i0.astype(jnp.float32)
                w_v[1, pl.ds(p, L)] = y - j0.astype(jnp.float32)
                w_v[2, pl.ds(p, L)] = z - k0.astype(jnp.float32)
                r = (i0 * G + j0) * G + k0
                b = g // (IDXB // L)
                o = p - b * IDXB
                for j, off in enumerate(_OFFS):
                    idx_v[j, b, pl.ds(o, L)] = r + off
                return c

            lax.fori_loop(0, GROUPS, pass_a, 0)

            def fire(b, c):
                for j in range(8):
                    pltpu.async_copy(
                        theta4.at[idx_v.at[j, b]],
                        rows_v.at[j, pl.ds(b * IDXB, IDXB)],
                        sem,
                    )
                return c

            lax.fori_loop(0, NB, fire, 0)

            def drain(b, c):
                for j in range(8):
                    pltpu.make_async_copy(
                        theta4.at[idx_v.at[j, b]],
                        rows_v.at[j, pl.ds(b * IDXB, IDXB)],
                        sem,
                    ).wait()
                return c

            lax.fori_loop(0, NB, drain, 0)

            def pass_b(g, c):
                p = g * L
                pidx = p + lanes
                wx = w_v[0, pl.ds(p, L)]
                wy = w_v[1, pl.ds(p, L)]
                wz = w_v[2, pl.ds(p, L)]
                ux, uy, uz = 1.0 - wx, 1.0 - wy, 1.0 - wz
                cw = []
                for (di, dj, dk) in _DIJK:
                    wgt = (wx if di else ux) * (wy if dj else uy)
                    cw.append(wgt * (wz if dk else uz))
                for ch in range(3):
                    csel = jnp.full((L,), ch, jnp.int32)
                    acc = None
                    for j in range(8):
                        jsel = jnp.full((L,), j, jnp.int32)
                        val = plsc.load_gather(rows_v, [jsel, pidx, csel])
                        term = cw[j] * val
                        acc = term if acc is None else acc + term
                    plsc.store_scatter(out_v, [3 * pidx + ch], acc)
                return c

            lax.fori_loop(0, GROUPS, pass_b, 0)
            pltpu.sync_copy(out_v, out.at[pl.ds(start * 3, CH * 3)])

        return carry

    lax.fori_loop(0, TPW, chunk_body, 0)


@jax.jit
def _run(coords, theta):
    theta4 = jnp.pad(theta, ((0, 0), (0, 0), (0, 0), (0, 1))).reshape(G * G * G, 4)
    coords_f = coords.reshape(-1)
    mesh = plsc.VectorSubcoreMesh(
        core_axis_name="c", subcore_axis_name="s", num_cores=NC, num_subcores=NS
    )
    out = pl.kernel(
        _body,
        out_type=jax.ShapeDtypeStruct((N * 3,), jnp.float32),
        mesh=mesh,
        compiler_params=pltpu.CompilerParams(
            needs_layout_passes=False, use_tc_tiling_on_sc=False
        ),
        scratch_types=[
            pltpu.VMEM((CH * 3,), jnp.float32),
            pltpu.VMEM((8, NB, IDXB), jnp.int32),
            pltpu.VMEM((8, CH, 4), jnp.float32),
            pltpu.VMEM((3, CH), jnp.float32),
            pltpu.VMEM((CH * 3,), jnp.float32),
            pltpu.SemaphoreType.DMA,
        ],
    )(theta4, coords_f)
    return out.reshape(N, 3)


def kernel(coords, theta):
    return _run(coords, theta)
